# baseline (speedup 1.0000x reference)
import math

import jax
import jax.numpy as jnp
from jax.experimental import pallas as pl
from jax.experimental.pallas import tpu as pltpu

_N = 10000
_E = 320000
_K = int(math.ceil(0.5 * _N))
_BLK = 1000

_W = 240
_NW = (_E + _W - 1) // _W
_EPAD = _NW * _W


def _scat_body(dst_ref, h_ref, out_ref, agg_ref, acc_ref, cur_ref):
    w = pl.program_id(0)

    @pl.when(w == 0)
    def _init():
        agg_ref[...] = jnp.zeros_like(agg_ref)
        acc_ref[...] = jnp.zeros_like(acc_ref)
        cur_ref[0] = -1

    base = w * _W

    def body(i, carry):
        acc, cur_d = carry
        e = base + i
        valid = e < _E
        d = jnp.where(valid, dst_ref[0, 0, i], -1)
        s = dst_ref[0, 1, i]
        row = h_ref[pl.ds(s, 1), :]
        flush = jnp.logical_and(cur_d >= 0, d != cur_d)

        @pl.when(flush)
        def _():
            agg_ref[pl.ds(cur_d, 1), :] += acc

        same = jnp.logical_and(valid, d == cur_d)
        acc = jnp.where(same, acc, jnp.zeros_like(acc)) + jnp.where(
            valid, row, jnp.zeros_like(row)
        )
        cur_d = jnp.where(valid, d, cur_d)
        cur_d = jnp.where(jnp.logical_or(valid, jnp.logical_not(flush)), cur_d, -1)
        return acc, cur_d

    acc, cur_d = jax.lax.fori_loop(
        0, _W, body, (acc_ref[...], cur_ref[0])
    )
    acc_ref[...] = acc
    cur_ref[0] = cur_d

    @pl.when(w == _NW - 1)
    def _emit():
        @pl.when(cur_d >= 0)
        def _final_flush():
            agg_ref[pl.ds(cur_d, 1), :] += acc

        out_ref[...] = agg_ref[...]


def _scatter(h, dst_sorted, src_sorted):
    pad = _EPAD - _E
    ds_p = jnp.concatenate([dst_sorted, jnp.zeros((pad,), jnp.int32)])
    sr_p = jnp.concatenate([src_sorted, jnp.zeros((pad,), jnp.int32)])
    idx = jnp.stack([ds_p, sr_p], axis=0).reshape(2, _NW, _W).transpose(1, 0, 2)
    return pl.pallas_call(
        _scat_body,
        grid=(_NW,),
        in_specs=[
            pl.BlockSpec((1, 2, _W), lambda w: (w, 0, 0), memory_space=pltpu.SMEM),
            pl.BlockSpec((_N, 128), lambda w: (0, 0)),
        ],
        out_specs=pl.BlockSpec((_N, 128), lambda w: (0, 0)),
        out_shape=jax.ShapeDtypeStruct((_N, 128), jnp.float32),
        scratch_shapes=[
            pltpu.VMEM((_N, 128), jnp.float32),
            pltpu.VMEM((1, 128), jnp.float32),
            pltpu.SMEM((1,), jnp.int32),
        ],
    )(idx, h)


def kernel(x, pos, edge_index, batch_idx, W_proj, b_proj, W_rel, b_rel, W_root):
    h = x @ W_proj.T + b_proj
    ei = edge_index.T
    src = ei[0]
    dst = ei[1]
    iota = jnp.arange(_E, dtype=jnp.int32)
    dst_sorted, perm = jax.lax.sort_key_val(dst, iota, is_stable=True)
    src_sorted = src[perm]
    agg = _scatter(h, dst_sorted, src_sorted)
    score = agg @ W_rel.T + b_rel + h @ W_root.T
    score = jnp.tanh(score.reshape(-1))
    top_scores, perm2 = jax.lax.top_k(score, _K)
    x_pool = h[perm2] * top_scores[:, None]
    batch_pool = batch_idx[perm2]
    return (x_pool, batch_pool)
